# baseline (speedup 1.0000x reference)
import functools
import math

import jax
import jax.numpy as jnp
import numpy as np
from jax.experimental import pallas as pl
from jax.experimental.pallas import tpu as pltpu

_S = 8
_LANES = 128


def _body(x8_ref, a_ref, b_ref, c_ref, out_ref):
    x8 = x8_ref[...]
    xb = x8.astype(jnp.bfloat16)
    x2b = xb * xb
    comps = (
        jax.lax.dot(x2b, a_ref[...], preferred_element_type=jnp.float32)
        + jax.lax.dot(xb, b_ref[...], preferred_element_type=jnp.float32)
        + c_ref[...]
    )
    m = comps
    for sh in (8, 16, 32, 64):
        m = jnp.maximum(m, pltpu.roll(m, sh, 1))
    e = jnp.exp(comps - m)
    s = e
    for sh in (8, 16, 32, 64):
        s = s + pltpu.roll(s, sh, 1)
    val = m + jnp.log(s)
    out_ref[...] = val[:, :_S]


def kernel(x, w, mu, sigma):
    N, D = x.shape
    K = w.shape[0]
    inv2 = 1.0 / (sigma * sigma)
    A = (-0.5 * inv2).T
    B = (mu * inv2).T
    c_k = (
        jnp.log(w[:, 0])
        - 0.5 * (D * math.log(2.0 * math.pi)
                 + jnp.sum(jnp.log(sigma * sigma), axis=1)
                 + jnp.sum(mu * mu * inv2, axis=1))
    )
    eye = jnp.eye(_S, dtype=jnp.float32)
    A3 = jnp.einsum("dk,st->sdkt", A, eye).reshape(_S * D, K * _S)
    B3 = jnp.einsum("dk,st->sdkt", B, eye).reshape(_S * D, K * _S)
    c3 = jnp.repeat(c_k, _S).reshape(1, _LANES)

    R = N // _S
    BR = 256
    x8 = x.reshape(R, _S * D)
    out = pl.pallas_call(
        _body,
        grid=(R // BR,),
        in_specs=[
            pl.BlockSpec((BR, _S * D), lambda i: (i, 0)),
            pl.BlockSpec((_S * D, _LANES), lambda i: (0, 0)),
            pl.BlockSpec((_S * D, _LANES), lambda i: (0, 0)),
            pl.BlockSpec((1, _LANES), lambda i: (0, 0)),
        ],
        out_specs=pl.BlockSpec((BR, _S), lambda i: (i, 0)),
        out_shape=jax.ShapeDtypeStruct((R, _S), jnp.float32),
        compiler_params=pltpu.CompilerParams(
            dimension_semantics=("arbitrary",),
        ),
    )(x8, A3.astype(jnp.bfloat16), B3.astype(jnp.bfloat16), c3)
    return out.reshape(N)


# speedup vs baseline: 1.8092x; 1.8092x over previous
import functools
import math

import jax
import jax.numpy as jnp
import numpy as np
from jax.experimental import pallas as pl
from jax.experimental.pallas import tpu as pltpu

_LOG2PI = math.log(2.0 * math.pi)
_CONTRACT_MINOR = (((1,), (1,)), ((), ()))


def _body(x_ref, w_ref, mu_ref, sigma_ref, out_ref, a_ref, b_ref, c_ref):
    K, D = mu_ref.shape

    @pl.when(pl.program_id(0) == 0)
    def _prep():
        sig = sigma_ref[...]
        inv2 = 1.0 / (sig * sig)
        a_ref[...] = (-0.5 * inv2).astype(jnp.bfloat16)
        b_ref[...] = (mu_ref[...] * inv2).astype(jnp.bfloat16)
        c_ref[...] = (
            jnp.log(w_ref[...])
            - 0.5 * (D * _LOG2PI
                     + jnp.sum(jnp.log(sig * sig), axis=1, keepdims=True)
                     + jnp.sum(mu_ref[...] * mu_ref[...] * inv2,
                               axis=1, keepdims=True))
        )

    xb = x_ref[...].astype(jnp.bfloat16)
    x2b = xb * xb
    comps = (
        jax.lax.dot_general(a_ref[...], x2b, _CONTRACT_MINOR,
                            preferred_element_type=jnp.float32)
        + jax.lax.dot_general(b_ref[...], xb, _CONTRACT_MINOR,
                              preferred_element_type=jnp.float32)
        + c_ref[...]
    )
    m = jnp.max(comps, axis=0, keepdims=True)
    s = jnp.sum(jnp.exp(comps - m), axis=0, keepdims=True)
    out_ref[...] = (m + jnp.log(s))[0]


def kernel(x, w, mu, sigma):
    N, D = x.shape
    K = w.shape[0]
    BN = 2048
    return pl.pallas_call(
        _body,
        grid=(N // BN,),
        in_specs=[
            pl.BlockSpec((BN, D), lambda i: (i, 0)),
            pl.BlockSpec((K, 1), lambda i: (0, 0)),
            pl.BlockSpec((K, D), lambda i: (0, 0)),
            pl.BlockSpec((K, D), lambda i: (0, 0)),
        ],
        out_specs=pl.BlockSpec((BN,), lambda i: (i,)),
        out_shape=jax.ShapeDtypeStruct((N,), jnp.float32),
        scratch_shapes=[
            pltpu.VMEM((K, D), jnp.bfloat16),
            pltpu.VMEM((K, D), jnp.bfloat16),
            pltpu.VMEM((K, 1), jnp.float32),
        ],
        compiler_params=pltpu.CompilerParams(
            dimension_semantics=("arbitrary",),
        ),
    )(x, w, mu, sigma)


# speedup vs baseline: 2.3480x; 1.2978x over previous
import functools
import math

import jax
import jax.numpy as jnp
import numpy as np
from jax.experimental import pallas as pl
from jax.experimental.pallas import tpu as pltpu

_LOG2PI = math.log(2.0 * math.pi)
_CONTRACT_MINOR = (((1,), (1,)), ((), ()))


def _body(x_ref, w_ref, mu_ref, sigma_ref, out_ref, a_ref, b_ref, c_ref):
    K, D = mu_ref.shape

    @pl.when(pl.program_id(0) == 0)
    def _prep():
        sig = sigma_ref[...]
        inv2 = 1.0 / (sig * sig)
        a_ref[...] = (-0.5 * inv2).astype(jnp.bfloat16)
        b_ref[...] = (mu_ref[...] * inv2).astype(jnp.bfloat16)
        c_ref[...] = (
            jnp.log(w_ref[...])
            - 0.5 * (D * _LOG2PI
                     + jnp.sum(jnp.log(sig * sig), axis=1, keepdims=True)
                     + jnp.sum(mu_ref[...] * mu_ref[...] * inv2,
                               axis=1, keepdims=True))
        )

    xb = x_ref[...].astype(jnp.bfloat16)
    x2b = xb * xb
    comps = (
        jax.lax.dot_general(a_ref[...], x2b, _CONTRACT_MINOR,
                            preferred_element_type=jnp.float32)
        + jax.lax.dot_general(b_ref[...], xb, _CONTRACT_MINOR,
                              preferred_element_type=jnp.float32)
        + c_ref[...]
    )
    m = jnp.max(comps, axis=0, keepdims=True)
    s = jnp.sum(jnp.exp(comps - m), axis=0, keepdims=True)
    out_ref[...] = (m + jnp.log(s))[0]


def kernel(x, w, mu, sigma):
    N, D = x.shape
    K = w.shape[0]
    BN = 8192
    return pl.pallas_call(
        _body,
        grid=(N // BN,),
        in_specs=[
            pl.BlockSpec((BN, D), lambda i: (i, 0)),
            pl.BlockSpec((K, 1), lambda i: (0, 0)),
            pl.BlockSpec((K, D), lambda i: (0, 0)),
            pl.BlockSpec((K, D), lambda i: (0, 0)),
        ],
        out_specs=pl.BlockSpec((BN,), lambda i: (i,)),
        out_shape=jax.ShapeDtypeStruct((N,), jnp.float32),
        scratch_shapes=[
            pltpu.VMEM((K, D), jnp.bfloat16),
            pltpu.VMEM((K, D), jnp.bfloat16),
            pltpu.VMEM((K, 1), jnp.float32),
        ],
        compiler_params=pltpu.CompilerParams(
            dimension_semantics=("arbitrary",),
        ),
    )(x, w, mu, sigma)


# speedup vs baseline: 2.4321x; 1.0358x over previous
import functools
import math

import jax
import jax.numpy as jnp
import numpy as np
from jax.experimental import pallas as pl
from jax.experimental.pallas import tpu as pltpu

_LOG2PI = math.log(2.0 * math.pi)
_CONTRACT_MINOR = (((1,), (1,)), ((), ()))


def _body(x_ref, w_ref, mu_ref, sigma_ref, out_ref, a_ref, b_ref, c_ref):
    K, D = mu_ref.shape

    @pl.when(pl.program_id(0) == 0)
    def _prep():
        sig = sigma_ref[...]
        inv2 = 1.0 / (sig * sig)
        a_ref[...] = (-0.5 * inv2).astype(jnp.bfloat16)
        b_ref[...] = (mu_ref[...] * inv2).astype(jnp.bfloat16)
        c_ref[...] = (
            jnp.log(w_ref[...])
            - 0.5 * (D * _LOG2PI
                     + jnp.sum(jnp.log(sig * sig), axis=1, keepdims=True)
                     + jnp.sum(mu_ref[...] * mu_ref[...] * inv2,
                               axis=1, keepdims=True))
        )

    xb = x_ref[...].astype(jnp.bfloat16)
    x2b = xb * xb
    comps = (
        jax.lax.dot_general(a_ref[...], x2b, _CONTRACT_MINOR,
                            preferred_element_type=jnp.float32)
        + jax.lax.dot_general(b_ref[...], xb, _CONTRACT_MINOR,
                              preferred_element_type=jnp.float32)
        + c_ref[...]
    )
    m = jnp.max(comps, axis=0, keepdims=True)
    s = jnp.sum(jnp.exp(comps - m), axis=0, keepdims=True)
    out_ref[...] = (m + jnp.log(s))[0]


def kernel(x, w, mu, sigma):
    N, D = x.shape
    K = w.shape[0]
    BN = 16384
    return pl.pallas_call(
        _body,
        grid=(N // BN,),
        in_specs=[
            pl.BlockSpec((BN, D), lambda i: (i, 0)),
            pl.BlockSpec((K, 1), lambda i: (0, 0)),
            pl.BlockSpec((K, D), lambda i: (0, 0)),
            pl.BlockSpec((K, D), lambda i: (0, 0)),
        ],
        out_specs=pl.BlockSpec((BN,), lambda i: (i,)),
        out_shape=jax.ShapeDtypeStruct((N,), jnp.float32),
        scratch_shapes=[
            pltpu.VMEM((K, D), jnp.bfloat16),
            pltpu.VMEM((K, D), jnp.bfloat16),
            pltpu.VMEM((K, 1), jnp.float32),
        ],
        compiler_params=pltpu.CompilerParams(
            dimension_semantics=("arbitrary",),
        ),
    )(x, w, mu, sigma)
